# baseline (speedup 1.0000x reference)
import jax
import jax.numpy as jnp
from jax.experimental import pallas as pl
from jax.experimental.pallas import tpu as pltpu

_H = 24
_W = 8
_HW = _H * _W


def _body(keep_n, x_ref, o_ref):
    xb = x_ref[0]
    colsum = jnp.sum(xb * xb, axis=0, keepdims=True)
    e = colsum[:, :_HW] + colsum[:, _HW:]
    lane = jax.lax.broadcasted_iota(jnp.int32, (_H, _HW), 1)
    rowi = jax.lax.broadcasted_iota(jnp.int32, (_H, _HW), 0)
    lrow = lane // _W
    cond = lrow == rowi
    g = jnp.where(cond, jnp.broadcast_to(e, (_H, _HW)), -1.0)
    rm_col = jnp.max(g, axis=1, keepdims=True)
    rmb = jnp.sum(jnp.where(cond, jnp.broadcast_to(rm_col, (_H, _HW)), 0.0),
                  axis=0, keepdims=True)
    less = rmb < rm_col
    eq_lower = (rmb == rm_col) & (lrow < rowi)
    cnt = jnp.sum(less.astype(jnp.float32) + eq_lower.astype(jnp.float32),
                  axis=1, keepdims=True)
    rank = cnt * (1.0 / _W)
    keep = (rank < keep_n).astype(xb.dtype)
    mask = jnp.sum(jnp.where(cond, jnp.broadcast_to(keep, (_H, _HW)), 0.0),
                   axis=0, keepdims=True)
    mask2 = jnp.concatenate([mask, mask], axis=1)
    o_ref[0] = xb * mask2


def kernel(x):
    b, c, h, w = x.shape
    rh = int(round(0.33 * h))
    keep_n = h - rh
    rows = c * h * w // 384
    x3 = x.reshape(b, rows, 384)
    out = pl.pallas_call(
        lambda x_ref, o_ref: _body(keep_n, x_ref, o_ref),
        grid=(b,),
        in_specs=[pl.BlockSpec((1, rows, 384), lambda i: (i, 0, 0))],
        out_specs=pl.BlockSpec((1, rows, 384), lambda i: (i, 0, 0)),
        out_shape=jax.ShapeDtypeStruct((b, rows, 384), x.dtype),
        compiler_params=pltpu.CompilerParams(
            dimension_semantics=("parallel",)),
    )(x3)
    return out.reshape(b, c, h, w)
